# baseline (speedup 1.0000x reference)
import functools

import jax
import jax.numpy as jnp
from jax import lax
from jax.experimental import pallas as pl
from jax.experimental.pallas import tpu as pltpu
from jax.experimental.pallas import tpu_sc as plsc

N = 10000
NP = 10240
E = 320000
D = 128
DC = 48
NC = 2
NS = 16
NW = NC * NS
EPW = E // NW
CSZ = 50
NCH = EPW // CSZ
RPT = NP // NS

_mesh = plsc.VectorSubcoreMesh(
    core_axis_name="c", subcore_axis_name="s", num_cores=NC, num_subcores=NS
)
_sc_params = pltpu.CompilerParams(use_tc_tiling_on_sc=False)


def _zero_rows(ref, nrows, ncols):
    z = jnp.zeros((16,), jnp.float32)

    def body(i, _):
        for k in range(ncols // 16):
            ref[i, 16 * k:16 * (k + 1)] = z
        return 0

    lax.fori_loop(0, nrows, body, 0)


@functools.partial(
    pl.kernel,
    out_type=(
        jax.ShapeDtypeStruct((NC * NP, 16), jnp.float32),
        jax.ShapeDtypeStruct((NC * NP, 16), jnp.float32),
    ),
    mesh=_mesh,
    compiler_params=_sc_params,
    scratch_types=[
        pltpu.VMEM_SHARED((NP, 16), jnp.float32),
        pltpu.VMEM_SHARED((NP, 16), jnp.float32),
        pltpu.VMEM((NCH, CSZ), jnp.int32),
        pltpu.VMEM((NCH, CSZ), jnp.int32),
        pltpu.VMEM((CSZ, 16), jnp.float32),
        pltpu.VMEM((RPT, 16), jnp.float32),
        pltpu.SemaphoreType.DMA,
    ],
)
def _deg_kernel(e_hbm, od_hbm, id_hbm, od_sp, id_sp, sidx, didx,
                ones_v, zb, sem):
    c = lax.axis_index("c")
    s = lax.axis_index("s")
    wid = c * NS + s

    ones = jnp.full((16,), 1.0, jnp.float32)

    def fill_ones(i, _):
        ones_v[i, :] = ones
        return 0

    lax.fori_loop(0, CSZ, fill_ones, 0)
    _zero_rows(zb, RPT, 16)
    pltpu.sync_copy(zb, od_sp.at[pl.ds(s * RPT, RPT)])
    pltpu.sync_copy(zb, id_sp.at[pl.ds(s * RPT, RPT)])
    pltpu.sync_copy(e_hbm.at[0, wid], sidx)
    pltpu.sync_copy(e_hbm.at[1, wid], didx)
    plsc.subcore_barrier()

    GRP = 5

    def body(g, _):
        for k in range(GRP):
            i = g * GRP + k
            pltpu.async_copy(ones_v, od_sp.at[sidx.at[i]], sem, add=True)
            pltpu.async_copy(ones_v, id_sp.at[didx.at[i]], sem, add=True)

        @pl.when(g > 0)
        def _():
            for k in range(2 * GRP):
                pltpu.make_async_copy(ones_v, od_sp.at[sidx.at[0]],
                                      sem).wait()
        return 0

    lax.fori_loop(0, NCH // GRP, body, 0)
    for k in range(2 * GRP):
        pltpu.make_async_copy(ones_v, od_sp.at[sidx.at[0]], sem).wait()
    plsc.subcore_barrier()

    base = c * NP + s * RPT
    pltpu.sync_copy(od_sp.at[pl.ds(s * RPT, RPT)], od_hbm.at[pl.ds(base, RPT)])
    pltpu.sync_copy(id_sp.at[pl.ds(s * RPT, RPT)], id_hbm.at[pl.ds(base, RPT)])


def _make_agg(width, nbuf, csz):
    nch = EPW // csz
    assert nch * csz == EPW and nch % nbuf == 0
    nblk = nch // nbuf
    zrows = 40 if csz < 80 else 80
    assert RPT % zrows == 0

    @functools.partial(
        pl.kernel,
        out_type=jax.ShapeDtypeStruct((NC * NP, width), jnp.float32),
        mesh=_mesh,
        compiler_params=_sc_params,
        scratch_types=[
            pltpu.VMEM_SHARED((NP, width), jnp.float32),
            pltpu.VMEM((nch, csz), jnp.int32),
            pltpu.VMEM((nch, csz), jnp.int32),
            [pltpu.VMEM((csz, width), jnp.float32) for _ in range(nbuf)],
            pltpu.SemaphoreType.DMA,
            [pltpu.SemaphoreType.DMA for _ in range(nbuf)],
        ],
    )
    def agg(h_hbm, e_hbm, out_hbm, acc_sp, sidx, didx, rows, sem, sems):
        c = lax.axis_index("c")
        s = lax.axis_index("s")
        wid = c * NS + s

        _zero_rows(rows[0], zrows, width)
        for k in range(RPT // zrows):
            pltpu.sync_copy(rows[0].at[pl.ds(0, zrows)],
                            acc_sp.at[pl.ds(s * RPT + k * zrows, zrows)])
        pltpu.sync_copy(e_hbm.at[0, wid], sidx)
        pltpu.sync_copy(e_hbm.at[1, wid], didx)
        plsc.subcore_barrier()

        for b in range(nbuf):
            pltpu.async_copy(h_hbm.at[sidx.at[b]], rows[b], sem)

        def body(j, _):
            for b in range(nbuf):
                i = j * nbuf + b
                pb = (b - 1) % nbuf
                ip = i - 1

                @pl.when(ip >= 0)
                def _():
                    pltpu.make_async_copy(rows[pb], acc_sp.at[didx.at[0]],
                                          sems[pb]).wait()

                @pl.when((ip >= 0) & (ip + nbuf < nch))
                def _():
                    pltpu.async_copy(h_hbm.at[sidx.at[ip + nbuf]], rows[pb],
                                     sem)

                pltpu.make_async_copy(h_hbm.at[sidx.at[i]], rows[b],
                                      sem).wait()
                pltpu.async_copy(rows[b], acc_sp.at[didx.at[i]], sems[b],
                                 add=True)
            return 0

        lax.fori_loop(0, nblk, body, 0)
        pltpu.make_async_copy(rows[(nch - 1) % nbuf], acc_sp.at[didx.at[0]],
                              sems[(nch - 1) % nbuf]).wait()
        plsc.subcore_barrier()

        base = c * NP + s * RPT
        pltpu.sync_copy(acc_sp.at[pl.ds(s * RPT, RPT)],
                        out_hbm.at[pl.ds(base, RPT)])

    return agg


_agg128 = _make_agg(D, 4, CSZ)
_agg48 = _make_agg(DC, 8, CSZ)


_R = 1000


def _scale(deg_ref):
    d = deg_ref[0, :, 0:1] + deg_ref[1, :, 0:1]
    return 1.0 / jnp.sqrt(jnp.maximum(d, 1.0))


def _mm1_body(od_ref, x_ref, w_ref, o_ref):
    o_ref[...] = jnp.dot(x_ref[...] * _scale(od_ref), w_ref[...],
                         preferred_element_type=jnp.float32)


def _mm2_body(p_ref, id_ref, od_ref, b1_ref, w2_ref, o_ref):
    agg = p_ref[0] + p_ref[1]
    x1 = jnp.maximum(agg * _scale(id_ref) + b1_ref[...], 0.0)
    o_ref[...] = jnp.dot(x1 * _scale(od_ref), w2_ref[...],
                         preferred_element_type=jnp.float32)


def _fin_body(q_ref, id_ref, b2_ref, o_ref):
    o_ref[...] = ((q_ref[0, :, :40] + q_ref[1, :, :40]) * _scale(id_ref)
                  + b2_ref[...])


def _deg_spec():
    return pl.BlockSpec((NC, _R, 16), lambda i: (0, i, 0))


def _mm1(od, x, w1):
    return pl.pallas_call(
        _mm1_body,
        grid=(N // _R,),
        in_specs=[
            _deg_spec(),
            pl.BlockSpec((_R, D), lambda i: (i, 0)),
            pl.BlockSpec((D, D), lambda i: (0, 0)),
        ],
        out_specs=pl.BlockSpec((_R, D), lambda i: (i, 0)),
        out_shape=jax.ShapeDtypeStruct((N, D), jnp.float32),
    )(od, x, w1)


def _mm2(p, id_, od, b1, w2p):
    return pl.pallas_call(
        _mm2_body,
        grid=(N // _R,),
        in_specs=[
            pl.BlockSpec((NC, _R, D), lambda i: (0, i, 0)),
            _deg_spec(),
            _deg_spec(),
            pl.BlockSpec((1, D), lambda i: (0, 0)),
            pl.BlockSpec((D, DC), lambda i: (0, 0)),
        ],
        out_specs=pl.BlockSpec((_R, DC), lambda i: (i, 0)),
        out_shape=jax.ShapeDtypeStruct((N, DC), jnp.float32),
    )(p, id_, od, b1, w2p)


def _fin(q, id_, b2):
    return pl.pallas_call(
        _fin_body,
        grid=(N // _R,),
        in_specs=[
            pl.BlockSpec((NC, _R, DC), lambda i: (0, i, 0)),
            _deg_spec(),
            pl.BlockSpec((1, 40), lambda i: (0, 0)),
        ],
        out_specs=pl.BlockSpec((_R, 40), lambda i: (i, 0)),
        out_shape=jax.ShapeDtypeStruct((N, 40), jnp.float32),
    )(q, id_, b2)


@jax.jit
def kernel(features, edge_index, W1, b1, W2, b2):
    er = edge_index.reshape(2, NW, NCH, CSZ)

    od_p, id_p = _deg_kernel(er)
    od = od_p.reshape(NC, NP, 16)
    id_ = id_p.reshape(NC, NP, 16)

    h1 = _mm1(od, features, W1)
    p1 = _agg128(h1, er).reshape(NC, NP, D)

    w2p = jnp.pad(W2, ((0, 0), (0, DC - W2.shape[1])))
    b1r = b1.reshape(1, D)

    h2 = _mm2(p1, id_, od, b1r, w2p)
    p2 = _agg48(h2, er).reshape(NC, NP, DC)

    return _fin(p2, id_, b2.reshape(1, 40))


# speedup vs baseline: 1.0026x; 1.0026x over previous
import functools

import jax
import jax.numpy as jnp
from jax import lax
from jax.experimental import pallas as pl
from jax.experimental.pallas import tpu as pltpu
from jax.experimental.pallas import tpu_sc as plsc

N = 10000
NP = 10240
E = 320000
D = 128
DC = 48
NC = 2
NS = 16
NW = NC * NS
EPW = E // NW
CSZ = 50
NCH = EPW // CSZ
RPT = NP // NS

_mesh = plsc.VectorSubcoreMesh(
    core_axis_name="c", subcore_axis_name="s", num_cores=NC, num_subcores=NS
)
_sc_params = pltpu.CompilerParams(use_tc_tiling_on_sc=False)


def _zero_rows(ref, nrows, ncols):
    z = jnp.zeros((16,), jnp.float32)

    def body(i, _):
        for k in range(ncols // 16):
            ref[i, 16 * k:16 * (k + 1)] = z
        return 0

    lax.fori_loop(0, nrows, body, 0)


@functools.partial(
    pl.kernel,
    out_type=(
        jax.ShapeDtypeStruct((NC * NP, 16), jnp.float32),
        jax.ShapeDtypeStruct((NC * NP, 16), jnp.float32),
    ),
    mesh=_mesh,
    compiler_params=_sc_params,
    scratch_types=[
        pltpu.VMEM_SHARED((NP, 16), jnp.float32),
        pltpu.VMEM_SHARED((NP, 16), jnp.float32),
        pltpu.VMEM((NCH, CSZ), jnp.int32),
        pltpu.VMEM((NCH, CSZ), jnp.int32),
        pltpu.VMEM((CSZ, 16), jnp.float32),
        pltpu.VMEM((RPT, 16), jnp.float32),
        pltpu.SemaphoreType.DMA,
    ],
)
def _deg_kernel(e_hbm, od_hbm, id_hbm, od_sp, id_sp, sidx, didx,
                ones_v, zb, sem):
    c = lax.axis_index("c")
    s = lax.axis_index("s")
    wid = c * NS + s

    ones = jnp.full((16,), 1.0, jnp.float32)

    def fill_ones(i, _):
        ones_v[i, :] = ones
        return 0

    lax.fori_loop(0, CSZ, fill_ones, 0)
    _zero_rows(zb, RPT, 16)
    pltpu.sync_copy(zb, od_sp.at[pl.ds(s * RPT, RPT)])
    pltpu.sync_copy(zb, id_sp.at[pl.ds(s * RPT, RPT)])
    pltpu.sync_copy(e_hbm.at[0, wid], sidx)
    pltpu.sync_copy(e_hbm.at[1, wid], didx)
    plsc.subcore_barrier()

    GRP = 5

    def body(g, _):
        for k in range(GRP):
            i = g * GRP + k
            pltpu.async_copy(ones_v, od_sp.at[sidx.at[i]], sem, add=True)
            pltpu.async_copy(ones_v, id_sp.at[didx.at[i]], sem, add=True)

        @pl.when(g > 0)
        def _():
            for k in range(2 * GRP):
                pltpu.make_async_copy(ones_v, od_sp.at[sidx.at[0]],
                                      sem).wait()
        return 0

    lax.fori_loop(0, NCH // GRP, body, 0)
    for k in range(2 * GRP):
        pltpu.make_async_copy(ones_v, od_sp.at[sidx.at[0]], sem).wait()
    plsc.subcore_barrier()

    base = c * NP + s * RPT
    pltpu.sync_copy(od_sp.at[pl.ds(s * RPT, RPT)], od_hbm.at[pl.ds(base, RPT)])
    pltpu.sync_copy(id_sp.at[pl.ds(s * RPT, RPT)], id_hbm.at[pl.ds(base, RPT)])


def _make_agg(width, nbuf, csz):
    nch = EPW // csz
    assert nch * csz == EPW and nch % nbuf == 0
    nblk = nch // nbuf
    zrows = 40 if csz < 80 else 80
    assert RPT % zrows == 0

    @functools.partial(
        pl.kernel,
        out_type=jax.ShapeDtypeStruct((NC * NP, width), jnp.float32),
        mesh=_mesh,
        compiler_params=_sc_params,
        scratch_types=[
            pltpu.VMEM_SHARED((NP, width), jnp.float32),
            pltpu.VMEM((nch, csz), jnp.int32),
            pltpu.VMEM((nch, csz), jnp.int32),
            [pltpu.VMEM((csz, width), jnp.float32) for _ in range(nbuf)],
            pltpu.SemaphoreType.DMA,
        ],
    )
    def agg(h_hbm, e_hbm, out_hbm, acc_sp, sidx, didx, rows, sem):
        c = lax.axis_index("c")
        s = lax.axis_index("s")
        wid = c * NS + s

        _zero_rows(rows[0], zrows, width)
        for k in range(RPT // zrows):
            pltpu.sync_copy(rows[0].at[pl.ds(0, zrows)],
                            acc_sp.at[pl.ds(s * RPT + k * zrows, zrows)])
        pltpu.sync_copy(e_hbm.at[0, wid], sidx)
        pltpu.sync_copy(e_hbm.at[1, wid], didx)
        plsc.subcore_barrier()

        for b in range(nbuf):
            pltpu.async_copy(h_hbm.at[sidx.at[b]], rows[b], sem)

        def body(j, _):
            for b in range(nbuf):
                i = j * nbuf + b
                pltpu.make_async_copy(h_hbm.at[sidx.at[i]], rows[b],
                                      sem).wait()
                pltpu.sync_copy(rows[b], acc_sp.at[didx.at[i]], add=True)

                @pl.when(j < nblk - 1)
                def _():
                    pltpu.async_copy(h_hbm.at[sidx.at[i + nbuf]], rows[b],
                                     sem)
            return 0

        lax.fori_loop(0, nblk, body, 0)
        plsc.subcore_barrier()

        base = c * NP + s * RPT
        pltpu.sync_copy(acc_sp.at[pl.ds(s * RPT, RPT)],
                        out_hbm.at[pl.ds(base, RPT)])

    return agg


_agg128 = _make_agg(D, 4, CSZ)
_agg48 = _make_agg(DC, 8, CSZ)


_R = 1000


def _scale(deg_ref):
    d = deg_ref[0, :, 0:1] + deg_ref[1, :, 0:1]
    return 1.0 / jnp.sqrt(jnp.maximum(d, 1.0))


def _mm1_body(od_ref, x_ref, w_ref, o_ref):
    o_ref[...] = jnp.dot(x_ref[...] * _scale(od_ref), w_ref[...],
                         preferred_element_type=jnp.float32)


def _mm2_body(p_ref, id_ref, od_ref, b1_ref, w2_ref, o_ref):
    agg = p_ref[0] + p_ref[1]
    x1 = jnp.maximum(agg * _scale(id_ref) + b1_ref[...], 0.0)
    o_ref[...] = jnp.dot(x1 * _scale(od_ref), w2_ref[...],
                         preferred_element_type=jnp.float32)


def _fin_body(q_ref, id_ref, b2_ref, o_ref):
    o_ref[...] = ((q_ref[0, :, :40] + q_ref[1, :, :40]) * _scale(id_ref)
                  + b2_ref[...])


def _deg_spec():
    return pl.BlockSpec((NC, _R, 16), lambda i: (0, i, 0))


def _mm1(od, x, w1):
    return pl.pallas_call(
        _mm1_body,
        grid=(N // _R,),
        in_specs=[
            _deg_spec(),
            pl.BlockSpec((_R, D), lambda i: (i, 0)),
            pl.BlockSpec((D, D), lambda i: (0, 0)),
        ],
        out_specs=pl.BlockSpec((_R, D), lambda i: (i, 0)),
        out_shape=jax.ShapeDtypeStruct((N, D), jnp.float32),
    )(od, x, w1)


def _mm2(p, id_, od, b1, w2p):
    return pl.pallas_call(
        _mm2_body,
        grid=(N // _R,),
        in_specs=[
            pl.BlockSpec((NC, _R, D), lambda i: (0, i, 0)),
            _deg_spec(),
            _deg_spec(),
            pl.BlockSpec((1, D), lambda i: (0, 0)),
            pl.BlockSpec((D, DC), lambda i: (0, 0)),
        ],
        out_specs=pl.BlockSpec((_R, DC), lambda i: (i, 0)),
        out_shape=jax.ShapeDtypeStruct((N, DC), jnp.float32),
    )(p, id_, od, b1, w2p)


def _fin(q, id_, b2):
    return pl.pallas_call(
        _fin_body,
        grid=(N // _R,),
        in_specs=[
            pl.BlockSpec((NC, _R, DC), lambda i: (0, i, 0)),
            _deg_spec(),
            pl.BlockSpec((1, 40), lambda i: (0, 0)),
        ],
        out_specs=pl.BlockSpec((_R, 40), lambda i: (i, 0)),
        out_shape=jax.ShapeDtypeStruct((N, 40), jnp.float32),
    )(q, id_, b2)


@jax.jit
def kernel(features, edge_index, W1, b1, W2, b2):
    er = edge_index.reshape(2, NW, NCH, CSZ)

    od_p, id_p = _deg_kernel(er)
    od = od_p.reshape(NC, NP, 16)
    id_ = id_p.reshape(NC, NP, 16)

    h1 = _mm1(od, features, W1)
    p1 = _agg128(h1, er).reshape(NC, NP, D)

    w2p = jnp.pad(W2, ((0, 0), (0, DC - W2.shape[1])))
    b1r = b1.reshape(1, D)

    h2 = _mm2(p1, id_, od, b1r, w2p)
    p2 = _agg48(h2, er).reshape(NC, NP, DC)

    return _fin(p2, id_, b2.reshape(1, 40))
